# baseline (speedup 1.0000x reference)
import jax
import jax.numpy as jnp
from jax.experimental import pallas as pl
from jax.experimental.pallas import tpu as pltpu

B = 8
T_AUDIO = 160000
HOP = 320
D = 256
K = 1024
T = T_AUDIO // HOP
TP = 512
TB = 4
R = TB * T


def _fused_kernel(frames_ref, w_ref, cb_ref, out_ref, ct_ref, csq_ref):
    b = pl.program_id(0)

    @pl.when(b == 0)
    def _init():
        ct = jnp.transpose(cb_ref[...])
        ct_ref[...] = ct.astype(jnp.bfloat16)
        csq_ref[...] = jnp.sum(ct * ct, axis=0, keepdims=True)

    frames = frames_ref[0]
    z = jnp.dot(frames, w_ref[...], preferred_element_type=jnp.float32)
    zb = z.astype(jnp.bfloat16)
    dots = jnp.dot(zb, ct_ref[...], preferred_element_type=jnp.float32)
    z_sq = jnp.sum(z * z, axis=1, keepdims=True)
    dist = (z_sq - 2.0 * dots) + csq_ref[...]

    codes = jnp.argmin(dist, axis=1)
    codes_row = codes.reshape(1, R).astype(jnp.int32)

    oh = (jax.lax.broadcasted_iota(jnp.int32, (K, R), 0) == codes_row)
    oh = oh.astype(jnp.bfloat16)
    dec = jax.lax.dot_general(
        ct_ref[...], oh, (((1,), (0,)), ((), ())),
        preferred_element_type=jnp.float32)
    for i in range(TB):
        out_ref[i] = dec[:, i * T:(i + 1) * T]


def kernel(source, W_enc, codebook):
    frames = source.reshape(B // TB, R, HOP)
    return pl.pallas_call(
        _fused_kernel,
        grid=(B // TB,),
        in_specs=[
            pl.BlockSpec((1, R, HOP), lambda b: (b, 0, 0)),
            pl.BlockSpec((HOP, D), lambda b: (0, 0)),
            pl.BlockSpec((K, D), lambda b: (0, 0)),
        ],
        out_specs=pl.BlockSpec((TB, D, T), lambda b: (b, 0, 0)),
        out_shape=jax.ShapeDtypeStruct((B, D, T), jnp.float32),
        scratch_shapes=[
            pltpu.VMEM((D, K), jnp.bfloat16),
            pltpu.VMEM((1, K), jnp.float32),
        ],
        compiler_params=pltpu.CompilerParams(
            dimension_semantics=("arbitrary",),
        ),
    )(frames, W_enc, codebook)


# speedup vs baseline: 1.0606x; 1.0606x over previous
import jax
import jax.numpy as jnp
from jax.experimental import pallas as pl
from jax.experimental.pallas import tpu as pltpu

B = 8
T_AUDIO = 160000
HOP = 320
D = 256
K = 1024
T = T_AUDIO // HOP
TP = 512
TB = 4
R = TB * T


def _fused_kernel(frames_ref, w_ref, cb_ref, out_ref, ct_ref, csq_ref):
    b = pl.program_id(0)

    @pl.when(b == 0)
    def _init():
        ct = jnp.transpose(cb_ref[...])
        ct_ref[...] = ct.astype(jnp.bfloat16)
        csq_ref[...] = jnp.sum(ct * ct, axis=0, keepdims=True)

    frames = frames_ref[...].reshape(R, HOP)
    z = jnp.dot(frames, w_ref[...], preferred_element_type=jnp.float32)
    zb = z.astype(jnp.bfloat16)
    dots = jnp.dot(zb, ct_ref[...], preferred_element_type=jnp.float32)
    z_sq = jnp.sum(z * z, axis=1, keepdims=True)
    dist = (z_sq - 2.0 * dots) + csq_ref[...]

    codes = jnp.argmin(dist, axis=1)
    codes_row = codes.reshape(1, R).astype(jnp.int32)

    oh = (jax.lax.broadcasted_iota(jnp.int32, (K, R), 0) == codes_row)
    oh = oh.astype(jnp.bfloat16)
    dec = jax.lax.dot_general(
        ct_ref[...], oh, (((1,), (0,)), ((), ())),
        preferred_element_type=jnp.float32)
    for i in range(TB):
        out_ref[i] = dec[:, i * T:(i + 1) * T]


def kernel(source, W_enc, codebook):
    frames = source.reshape(B, T, HOP)
    return pl.pallas_call(
        _fused_kernel,
        grid=(B // TB,),
        in_specs=[
            pl.BlockSpec((TB, T, HOP), lambda b: (b, 0, 0)),
            pl.BlockSpec((HOP, D), lambda b: (0, 0)),
            pl.BlockSpec((K, D), lambda b: (0, 0)),
        ],
        out_specs=pl.BlockSpec((TB, D, T), lambda b: (b, 0, 0)),
        out_shape=jax.ShapeDtypeStruct((B, D, T), jnp.float32),
        scratch_shapes=[
            pltpu.VMEM((D, K), jnp.bfloat16),
            pltpu.VMEM((1, K), jnp.float32),
        ],
        compiler_params=pltpu.CompilerParams(
            dimension_semantics=("arbitrary",),
        ),
    )(frames, W_enc, codebook)
